# baseline (speedup 1.0000x reference)
import functools

import jax
import jax.numpy as jnp
from jax import lax
from jax.experimental import pallas as pl
from jax.experimental.pallas import tpu as pltpu
from jax.experimental.pallas import tpu_sc as plsc

B, S, D = 2, 2048, 768
E = 8
DF = 768
T = B * S
TILE = 128
PT = T + E * TILE
NT = PT // TILE
LB_ALPHA = 0.01
Z_ALPHA = 1e-4

CH = 512
NCH = T // CH

NC = 2
NS = 16
NW = NC * NS
TPW = T // NW
NCHK = 4
CW = TPW // NCHK


def _gate_body(x_ref, wg_ref, pos_ref, meta_ref, loss_ref, lg_scr):
    k = pl.program_id(0)
    lg_scr[pl.ds(k * CH, CH), :] = lax.dot_general(
        x_ref[...], wg_ref[...], (((1,), (1,)), ((), ())),
        preferred_element_type=jnp.float32)

    @pl.when(k == NCH - 1)
    def _():
        logits = lg_scr[...]
        m = jnp.max(logits, axis=1, keepdims=True)
        ex = jnp.exp(logits - m)
        se = jnp.sum(ex, axis=1, keepdims=True)
        scores = ex / se
        smax = jnp.max(scores, axis=1, keepdims=True)
        lane = lax.broadcasted_iota(jnp.int32, (T, E), 1)
        top = jnp.min(jnp.where(scores == smax, lane, E), axis=1,
                      keepdims=True)
        ohi = (lane == top).astype(jnp.int32)
        ohf = ohi.astype(jnp.float32)
        tri_le = (lax.broadcasted_iota(jnp.int32, (CH, CH), 1)
                  <= lax.broadcasted_iota(jnp.int32, (CH, CH), 0)
                  ).astype(jnp.float32)
        parts = [
            lax.dot_general(tri_le, ohf[kk * CH:(kk + 1) * CH, :],
                            (((1,), (0,)), ((), ())),
                            preferred_element_type=jnp.float32)
            for kk in range(NCH)
        ]
        segs = []
        carry = jnp.zeros((1, E), jnp.float32)
        for kk in range(NCH):
            segs.append(parts[kk] + carry)
            carry = carry + parts[kk][CH - 1:CH, :]
        c = jnp.concatenate(segs, axis=0).astype(jnp.int32)
        excl = c - ohi
        counts = c[T - 1:T, :]
        pc = ((counts + TILE - 1) // TILE) * TILE
        tri = (lax.broadcasted_iota(jnp.int32, (E, E), 0)
               < lax.broadcasted_iota(jnp.int32, (E, E), 1)
               ).astype(jnp.float32)
        pad_off = lax.dot_general(pc.astype(jnp.float32), tri,
                                  (((1,), (0,)), ((), ())),
                                  preferred_element_type=jnp.float32
                                  ).astype(jnp.int32)
        posv = jnp.sum(ohi * (pad_off + excl), axis=1)
        pos_ref[...] = posv.reshape(NW, TPW)
        eyeE = (lax.broadcasted_iota(jnp.int32, (E, E), 0)
                == lax.broadcasted_iota(jnp.int32, (E, E), 1)
                ).astype(jnp.float32)
        counts_col = lax.dot_general(eyeE, counts.astype(jnp.float32),
                                     (((1,), (1,)), ((), ())),
                                     preferred_element_type=jnp.float32)
        pc_col = ((counts_col.astype(jnp.int32) + TILE - 1) // TILE) * TILE
        triL = (lax.broadcasted_iota(jnp.int32, (E, E), 1)
                < lax.broadcasted_iota(jnp.int32, (E, E), 0)
                ).astype(jnp.float32)
        ts_col = (lax.dot_general(triL, pc_col.astype(jnp.float32),
                                  (((1,), (0,)), ((), ())),
                                  preferred_element_type=jnp.float32)
                  .astype(jnp.int32) // TILE)
        itE = lax.broadcasted_iota(jnp.int32, (E, NT), 1)
        eE = lax.broadcasted_iota(jnp.int32, (E, NT), 0)
        ecol = lax.broadcasted_iota(jnp.int32, (E, 1), 0)
        te_row = jnp.sum((itE >= ts_col).astype(jnp.int32), axis=0,
                         keepdims=True) - 1
        pres_col = counts_col > 0.5
        lp = jnp.max(jnp.where(pres_col, ecol, -1), axis=0, keepdims=True)
        te_row = jnp.minimum(te_row, lp)
        nxt_row = jnp.min(jnp.where((eE > te_row) & pres_col, eE, E),
                          axis=0, keepdims=True)
        nxt2_row = jnp.min(jnp.where((eE > nxt_row) & pres_col, eE, E),
                           axis=0, keepdims=True)
        par_row = jnp.sum((pres_col & (eE <= te_row)).astype(jnp.int32),
                          axis=0, keepdims=True) % 3
        meta_ref[...] = jnp.concatenate(
            [te_row, nxt_row, nxt2_row, par_row,
             jnp.zeros((E - 4, NT), jnp.int32)], axis=0)
        frac = counts.astype(jnp.float32) / T
        prob = jnp.sum(scores, axis=0, keepdims=True) / T
        lb = LB_ALPHA * E * jnp.sum(frac * prob)
        lse = m + jnp.log(se)
        z = Z_ALPHA * jnp.sum(lse * lse) / T
        loss_ref[...] = jnp.broadcast_to(lb + z, (1, 1))


def _gate(xf, Wg):
    return pl.pallas_call(
        _gate_body,
        grid=(NCH,),
        in_specs=[
            pl.BlockSpec((CH, D), lambda k: (k, 0)),
            pl.BlockSpec((E, D), lambda k: (0, 0)),
        ],
        out_specs=(
            pl.BlockSpec((NW, TPW), lambda k: (0, 0)),
            pl.BlockSpec((E, NT), lambda k: (0, 0)),
            pl.BlockSpec((1, 1), lambda k: (0, 0)),
        ),
        out_shape=(
            jax.ShapeDtypeStruct((NW, TPW), jnp.int32),
            jax.ShapeDtypeStruct((E, NT), jnp.int32),
            jax.ShapeDtypeStruct((1, 1), jnp.float32),
        ),
        scratch_shapes=[pltpu.VMEM((T, E), jnp.float32)],
        compiler_params=pltpu.CompilerParams(
            dimension_semantics=("arbitrary",)),
    )(xf, Wg)


@functools.lru_cache(maxsize=None)
def _sc_kernels():
    mesh = plsc.VectorSubcoreMesh(core_axis_name="c", subcore_axis_name="s",
                                  num_cores=NC, num_subcores=NS)
    scratch = [
        pltpu.VMEM((NCHK, CW), jnp.int32),
        pltpu.VMEM((TPW, D), jnp.float32),
        pltpu.SemaphoreType.DMA((NCHK,)),
        pltpu.SemaphoreType.DMA((NCHK,)),
    ]

    @functools.partial(
        pl.kernel, mesh=mesh,
        out_type=jax.ShapeDtypeStruct((PT, D), jnp.float32),
        scratch_types=scratch,
    )
    def permute_k(x_hbm, pos_hbm, xs_hbm, idx2, rows_v, sem_l, sem_s):
        wid = lax.axis_index("s") * NC + lax.axis_index("c")
        base = wid * TPW
        for c in range(NCHK):
            pltpu.sync_copy(pos_hbm.at[wid, pl.ds(c * CW, CW)], idx2.at[c])
        loads = [
            pltpu.async_copy(x_hbm.at[pl.ds(base + c * CW, CW)],
                             rows_v.at[pl.ds(c * CW, CW)], sem_l.at[c])
            for c in range(NCHK)
        ]
        stores = []
        for c in range(NCHK):
            loads[c].wait()
            stores.append(
                pltpu.async_copy(rows_v.at[pl.ds(c * CW, CW)],
                                 xs_hbm.at[idx2.at[c]], sem_s.at[c]))
        for st in stores:
            st.wait()

    @functools.partial(
        pl.kernel, mesh=mesh,
        out_type=jax.ShapeDtypeStruct((T, D), jnp.float32),
        scratch_types=scratch,
    )
    def unpermute_k(ys_hbm, pos_hbm, out_hbm, idx2, rows_v,
                    sem_l, sem_s):
        wid = lax.axis_index("s") * NC + lax.axis_index("c")
        base = wid * TPW
        for c in range(NCHK):
            pltpu.sync_copy(pos_hbm.at[wid, pl.ds(c * CW, CW)], idx2.at[c])
        gathers = [
            pltpu.async_copy(ys_hbm.at[idx2.at[c]],
                             rows_v.at[pl.ds(c * CW, CW)], sem_l.at[c])
            for c in range(NCHK)
        ]
        stores = []
        for c in range(NCHK):
            gathers[c].wait()
            stores.append(
                pltpu.async_copy(rows_v.at[pl.ds(c * CW, CW)],
                                 out_hbm.at[pl.ds(base + c * CW, CW)],
                                 sem_s.at[c]))
        for st in stores:
            st.wait()

    return permute_k, unpermute_k


def _permute(xf, pos):
    return _sc_kernels()[0](xf, pos)


def _unpermute(ys, pos):
    return _sc_kernels()[1](ys, pos)


def _ffn_body(meta_ref, xs_ref, w1_hbm, b1_ref, w2_hbm, b2_ref, out_ref,
              w1buf, w2buf, sem1, sem2):
    i = pl.program_id(0)
    e = meta_ref[0, i]
    nxt = meta_ref[1, i]
    nxt2 = meta_ref[2, i]
    slot = meta_ref[3, i]

    def cpy(eidx, s):
        return (pltpu.make_async_copy(w1_hbm.at[eidx], w1buf.at[s],
                                      sem1.at[s]),
                pltpu.make_async_copy(w2_hbm.at[eidx], w2buf.at[s],
                                      sem2.at[s]))

    @pl.when(i == 0)
    def _():
        c1, c2 = cpy(e, slot)
        c1.start()
        c2.start()

        @pl.when(nxt < E)
        def _():
            n1, n2 = cpy(nxt, (slot + 1) % 3)
            n1.start()
            n2.start()

    prev = meta_ref[0, jnp.maximum(i - 1, 0)]
    boundary = (i == 0) | (e != prev)

    @pl.when(boundary)
    def _():
        c1, c2 = cpy(e, slot)
        c1.wait()
        c2.wait()

        @pl.when(nxt2 < E)
        def _():
            n1, n2 = cpy(nxt2, (slot + 2) % 3)
            n1.start()
            n2.start()

    xv = xs_ref[...]
    h = jnp.dot(xv, w1buf[slot], preferred_element_type=jnp.float32)
    h = jnp.maximum(h + b1_ref[pl.ds(e, 1), :], 0.0)
    out_ref[...] = (jnp.dot(h, w2buf[slot], preferred_element_type=jnp.float32)
                    + b2_ref[pl.ds(e, 1), :])


def _ffn(meta, xs, w1, b1, w2, b2):
    return pl.pallas_call(
        _ffn_body,
        grid=(NT,),
        in_specs=[
            pl.BlockSpec(memory_space=pltpu.SMEM),
            pl.BlockSpec((TILE, D), lambda i: (i, 0)),
            pl.BlockSpec(memory_space=pl.ANY),
            pl.BlockSpec((E, DF), lambda i: (0, 0)),
            pl.BlockSpec(memory_space=pl.ANY),
            pl.BlockSpec((E, D), lambda i: (0, 0)),
        ],
        out_specs=pl.BlockSpec((TILE, D), lambda i: (i, 0)),
        out_shape=jax.ShapeDtypeStruct((PT, D), jnp.float32),
        scratch_shapes=[
            pltpu.VMEM((3, D, DF), jnp.float32),
            pltpu.VMEM((3, DF, D), jnp.float32),
            pltpu.SemaphoreType.DMA((3,)),
            pltpu.SemaphoreType.DMA((3,)),
        ],
        compiler_params=pltpu.CompilerParams(
            dimension_semantics=("arbitrary",)),
    )(meta, xs, w1, b1, w2, b2)


def kernel(x, Wg, w1, b1, w2, b2):
    xf = x.reshape(T, D)
    pos, meta, loss2 = _gate(xf, Wg)
    xs = _permute(xf, pos)
    ys = _ffn(meta, xs, w1, b1, w2, b2)
    outf = _unpermute(ys, pos)
    return outf.reshape(B, S, D), loss2[0, 0]
